# baseline (speedup 1.0000x reference)
import functools

import jax
import jax.numpy as jnp
from jax import lax
from jax.experimental import pallas as pl
from jax.experimental.pallas import tpu as pltpu
from jax.experimental.pallas import tpu_sc as plsc

NUM_IDS = 4096 * 200
EMBED = 64
G = 4
ROWW = EMBED * G
NGRP = NUM_IDS // G
NC, NS = 2, 16
NW = NC * NS
PER_W = NGRP // NW
BLK = 128
SLOTS = PER_W // BLK
RING = 2
L = 16


def _sc_body(ids_hbm, table_hbm, out_hbm, ids_v, idx_v, rows_v,
             gsem, ssem):
    sid = lax.axis_index("s")
    wid = sid * NC + lax.axis_index("c")
    base_w = wid * PER_W

    pltpu.sync_copy(ids_hbm.at[wid], ids_v)

    def compute(k, carry):
        w = ids_v[pl.ds(k * L, L)]
        va = w & 255
        vb = (w >> 8) & 255
        vc = (w >> 16) & 255
        vd = (w >> 24) & 255
        idx_v[pl.ds(k * L, L)] = ((va * 3 + vb) * 3 + vc) * 3 + vd
        return carry

    lax.fori_loop(0, PER_W // L, compute, 0)

    def gather(s, b):
        return pltpu.make_async_copy(
            table_hbm.at[idx_v.at[pl.ds(s * BLK, BLK)]],
            rows_v.at[b],
            gsem,
        )

    def store(s, b):
        return pltpu.make_async_copy(
            rows_v.at[b],
            out_hbm.at[pl.ds(base_w + s * BLK, BLK)],
            ssem,
        )

    def group(g, carry):
        for b in range(RING):
            s = g * RING + b

            @pl.when(s >= RING)
            def _wait_buffer_free():
                store(0, b).wait()

            gather(s, b).start()

            @pl.when(s >= 1)
            def _drain_prev_and_store():
                gather(0, 1 - b).wait()
                store(s - 1, 1 - b).start()

        return carry

    lax.fori_loop(0, SLOTS // RING, group, 0)

    b_last = (SLOTS - 1) % RING
    gather(0, b_last).wait()
    store(SLOTS - 1, b_last).start()
    store(0, 0).wait()
    store(0, 1).wait()


def kernel(modality_ids, modality_embedding):
    ids = lax.bitcast_convert_type(
        modality_ids.astype(jnp.int8).reshape(NW, PER_W, G), jnp.int32
    )
    t = modality_embedding
    t2 = jnp.concatenate(
        [jnp.repeat(t, 3, axis=0), jnp.tile(t, (3, 1))], axis=1
    )
    t4 = jnp.concatenate(
        [jnp.repeat(t2, 9, axis=0), jnp.tile(t2, (9, 1))], axis=1
    )

    mesh = plsc.VectorSubcoreMesh(core_axis_name="c", subcore_axis_name="s")
    run = functools.partial(
        pl.kernel,
        mesh=mesh,
        out_type=jax.ShapeDtypeStruct((NGRP, ROWW), jnp.float32),
        scratch_types=[
            pltpu.VMEM((PER_W,), jnp.int32),
            pltpu.VMEM((PER_W,), jnp.int32),
            pltpu.VMEM((RING, BLK, ROWW), jnp.float32),
            pltpu.SemaphoreType.DMA,
            pltpu.SemaphoreType.DMA,
        ],
    )(_sc_body)
    out = run(ids, t4)
    return out.reshape(modality_ids.shape + (EMBED,))


# speedup vs baseline: 1.4677x; 1.4677x over previous
import functools

import jax
import jax.numpy as jnp
from jax import lax
from jax.experimental import pallas as pl
from jax.experimental.pallas import tpu as pltpu
from jax.experimental.pallas import tpu_sc as plsc

NUM_IDS = 4096 * 200
EMBED = 64
G = 4
ROWW = EMBED * G
NGRP = NUM_IDS // G
NC, NS = 2, 16
NW = NC * NS
PER_W = NGRP // NW
BLK = 128
SLOTS = PER_W // BLK
RING = 2
L = 16


def _sc_body(ids_hbm, table_hbm, out_hbm, ids_v, idx_v, rows_v,
             gsem, ssem):
    sid = lax.axis_index("s")
    wid = sid * NC + lax.axis_index("c")
    base_w = wid * PER_W

    pltpu.sync_copy(ids_hbm.at[wid], ids_v)

    tbase = wid * 81

    def compute(k, carry):
        w = ids_v[pl.ds(k * L, L)]
        va = w & 255
        vb = (w >> 8) & 255
        vc = (w >> 16) & 255
        vd = (w >> 24) & 255
        idx_v[pl.ds(k * L, L)] = tbase + ((va * 3 + vb) * 3 + vc) * 3 + vd
        return carry

    lax.fori_loop(0, PER_W // L, compute, 0)

    def gather(s, b):
        return pltpu.make_async_copy(
            table_hbm.at[idx_v.at[pl.ds(s * BLK, BLK)]],
            rows_v.at[b],
            gsem,
        )

    def store(s, b):
        return pltpu.make_async_copy(
            rows_v.at[b],
            out_hbm.at[pl.ds(base_w + s * BLK, BLK)],
            ssem,
        )

    def group(g, carry):
        for b in range(RING):
            s = g * RING + b

            @pl.when(s >= RING)
            def _wait_buffer_free():
                store(0, b).wait()

            gather(s, b).start()

            @pl.when(s >= 1)
            def _drain_prev_and_store():
                gather(0, 1 - b).wait()
                store(s - 1, 1 - b).start()

        return carry

    lax.fori_loop(0, SLOTS // RING, group, 0)

    b_last = (SLOTS - 1) % RING
    gather(0, b_last).wait()
    store(SLOTS - 1, b_last).start()
    store(0, 0).wait()
    store(0, 1).wait()


def kernel(modality_ids, modality_embedding):
    ids = lax.bitcast_convert_type(
        modality_ids.astype(jnp.int8).reshape(NW, PER_W, G), jnp.int32
    )
    t = modality_embedding
    t2 = jnp.concatenate(
        [jnp.repeat(t, 3, axis=0), jnp.tile(t, (3, 1))], axis=1
    )
    t4 = jnp.concatenate(
        [jnp.repeat(t2, 9, axis=0), jnp.tile(t2, (9, 1))], axis=1
    )
    t4 = jnp.tile(t4, (NW, 1))

    mesh = plsc.VectorSubcoreMesh(core_axis_name="c", subcore_axis_name="s")
    run = functools.partial(
        pl.kernel,
        mesh=mesh,
        out_type=jax.ShapeDtypeStruct((NGRP, ROWW), jnp.float32),
        scratch_types=[
            pltpu.VMEM((PER_W,), jnp.int32),
            pltpu.VMEM((PER_W,), jnp.int32),
            pltpu.VMEM((RING, BLK, ROWW), jnp.float32),
            pltpu.SemaphoreType.DMA,
            pltpu.SemaphoreType.DMA,
        ],
    )(_sc_body)
    out = run(ids, t4)
    return out.reshape(modality_ids.shape + (EMBED,))
